# Initial kernel scaffold; baseline (speedup 1.0000x reference)
import jax
import jax.numpy as jnp
from jax.experimental import pallas as pl


def kernel(input_word_ids, emb_table, dense_w, dense_b):
    raise NotImplementedError("write your pallas kernel here")


# speedup vs baseline: 38.3684x; 38.3684x over previous
import functools

import jax
import jax.numpy as jnp
from jax import lax
from jax.experimental import pallas as pl
from jax.experimental.pallas import tpu as pltpu
from jax.experimental.pallas import tpu_sc as plsc

EMBED_DIM = 128
BATCH = 4096
SEQ = 200
HALF = SEQ // 2
NC = 2
NS = 16
NW = NC * NS
BPW = BATCH // NW
NCHUNK = EMBED_DIM // 16


def _shuffle(v, idx):
    dn = lax.GatherDimensionNumbers(
        offset_dims=(), collapsed_slice_dims=(0,), start_index_map=(0,))
    return lax.gather(v, idx[:, None], dn, slice_sizes=(1,),
                      mode=lax.GatherScatterMode.PROMISE_IN_BOUNDS)


def _sc_body(ids_hbm, table_hbm, w_hbm, b_hbm, out_hbm,
             idx_v, rows_v, w_v, b_v, out_v, sem):
    wid = lax.axis_index("s") * NC + lax.axis_index("c")
    base = wid * BPW

    pltpu.sync_copy(ids_hbm.at[pl.ds(base * 2, BPW * 2)], idx_v)
    pltpu.sync_copy(w_hbm, w_v)
    pltpu.sync_copy(b_hbm, b_v)
    bias = b_v[...]
    w_chunks = [w_v[pl.ds(16 * c, 16)] for c in range(NCHUNK)]

    lane = lax.broadcasted_iota(jnp.int32, (16,), 0)

    def group_body(g, carry):
        out_vec = jnp.full((16,), 0.0, dtype=jnp.float32)
        for j in range(16):
            i = g * 16 + j
            c0 = pltpu.async_copy(table_hbm.at[idx_v.at[2 * i]],
                                  rows_v.at[pl.ds(0, HALF)], sem)
            c1 = pltpu.async_copy(table_hbm.at[idx_v.at[2 * i + 1]],
                                  rows_v.at[pl.ds(HALF, HALF)], sem)
            c0.wait()
            c1.wait()

            def red_body(r, m):
                return tuple(
                    jnp.maximum(m[c], rows_v[r, pl.ds(16 * c, 16)])
                    for c in range(NCHUNK))

            neg = jnp.full((16,), -jnp.inf, dtype=jnp.float32)
            m = lax.fori_loop(0, SEQ, red_body, (neg,) * NCHUNK)

            p = m[0] * w_chunks[0]
            for c in range(1, NCHUNK):
                p = p + m[c] * w_chunks[c]
            for s in (8, 4, 2, 1):
                p = p + _shuffle(p, lane ^ s)
            out_vec = jnp.where(lane == j, p + bias, out_vec)
        out_v[pl.ds(g * 16, 16)] = out_vec
        return carry

    lax.fori_loop(0, BPW // 16, group_body, 0)
    pltpu.sync_copy(out_v, out_hbm.at[pl.ds(base, BPW)])


def kernel(input_word_ids, emb_table, dense_w, dense_b):
    ids2 = jnp.reshape(input_word_ids.astype(jnp.int32), (2 * BATCH, HALF))
    w_flat = jnp.reshape(dense_w, (EMBED_DIM,))
    mesh = plsc.VectorSubcoreMesh(core_axis_name="c", subcore_axis_name="s")
    run = functools.partial(
        pl.kernel,
        mesh=mesh,
        out_type=jax.ShapeDtypeStruct((BATCH,), jnp.float32),
        scratch_types=[
            pltpu.VMEM((2 * BPW, HALF), jnp.int32),
            pltpu.VMEM((SEQ, EMBED_DIM), jnp.float32),
            pltpu.VMEM((EMBED_DIM,), jnp.float32),
            pltpu.VMEM((16,), jnp.float32),
            pltpu.VMEM((BPW,), jnp.float32),
            pltpu.SemaphoreType.DMA,
        ],
    )(_sc_body)
    b16 = jnp.broadcast_to(jnp.reshape(dense_b, (1,)), (16,))
    out = run(ids2, emb_table, w_flat, b16)
    return jnp.reshape(out, (BATCH, 1))


# speedup vs baseline: 67.3638x; 1.7557x over previous
import functools

import jax
import jax.numpy as jnp
from jax import lax
from jax.experimental import pallas as pl
from jax.experimental.pallas import tpu as pltpu
from jax.experimental.pallas import tpu_sc as plsc

EMBED_DIM = 128
BATCH = 4096
SEQ = 200
HALF = SEQ // 2
NC = 2
NS = 16
NW = NC * NS
BPW = BATCH // NW
NCHUNK = EMBED_DIM // 16


def _shuffle(v, idx):
    dn = lax.GatherDimensionNumbers(
        offset_dims=(), collapsed_slice_dims=(0,), start_index_map=(0,))
    return lax.gather(v, idx[:, None], dn, slice_sizes=(1,),
                      mode=lax.GatherScatterMode.PROMISE_IN_BOUNDS)


def _sc_body(ids_hbm, table_hbm, w_hbm, b_hbm, out_hbm,
             idx_v, rows_v, w_v, b_v, out_v, sem0, sem1):
    wid = lax.axis_index("s") * NC + lax.axis_index("c")
    base = wid * BPW

    pltpu.sync_copy(ids_hbm.at[pl.ds(base * 2, BPW * 2)], idx_v)
    pltpu.sync_copy(w_hbm, w_v)
    pltpu.sync_copy(b_hbm, b_v)
    bias = b_v[...]
    w_chunks = [w_v[pl.ds(16 * c, 16)] for c in range(NCHUNK)]

    lane = lax.broadcasted_iota(jnp.int32, (16,), 0)
    sems = (sem0, sem1)

    def start_row(i, buf, sem):
        pltpu.async_copy(table_hbm.at[idx_v.at[2 * i]],
                         rows_v.at[buf, pl.ds(0, HALF)], sem)
        pltpu.async_copy(table_hbm.at[idx_v.at[2 * i + 1]],
                         rows_v.at[buf, pl.ds(HALF, HALF)], sem)

    def wait_row(i, buf, sem):
        pltpu.make_async_copy(table_hbm.at[idx_v.at[2 * i]],
                              rows_v.at[buf, pl.ds(0, HALF)], sem).wait()
        pltpu.make_async_copy(table_hbm.at[idx_v.at[2 * i + 1]],
                              rows_v.at[buf, pl.ds(HALF, HALF)], sem).wait()

    start_row(0, 0, sem0)

    def group_body(g, carry):
        out_vec = jnp.full((16,), 0.0, dtype=jnp.float32)
        for j in range(16):
            i = g * 16 + j
            buf, nbuf = j % 2, (j + 1) % 2

            @pl.when(i + 1 < BPW)
            def _():
                start_row(i + 1, nbuf, sems[nbuf])

            wait_row(i, buf, sems[buf])

            def red_body(r, m):
                return tuple(
                    jnp.maximum(m[c], rows_v[buf, r, pl.ds(16 * c, 16)])
                    for c in range(NCHUNK))

            neg = jnp.full((16,), -jnp.inf, dtype=jnp.float32)
            m = lax.fori_loop(0, SEQ, red_body, (neg,) * NCHUNK, unroll=4)

            p = m[0] * w_chunks[0]
            for c in range(1, NCHUNK):
                p = p + m[c] * w_chunks[c]
            for s in (8, 4, 2, 1):
                p = p + _shuffle(p, lane ^ s)
            out_vec = jnp.where(lane == j, p + bias, out_vec)
        out_v[pl.ds(g * 16, 16)] = out_vec
        return carry

    lax.fori_loop(0, BPW // 16, group_body, 0)
    pltpu.sync_copy(out_v, out_hbm.at[pl.ds(base, BPW)])


def kernel(input_word_ids, emb_table, dense_w, dense_b):
    ids2 = jnp.reshape(input_word_ids.astype(jnp.int32), (2 * BATCH, HALF))
    w_flat = jnp.reshape(dense_w, (EMBED_DIM,))
    mesh = plsc.VectorSubcoreMesh(core_axis_name="c", subcore_axis_name="s")
    run = functools.partial(
        pl.kernel,
        mesh=mesh,
        out_type=jax.ShapeDtypeStruct((BATCH,), jnp.float32),
        scratch_types=[
            pltpu.VMEM((2 * BPW, HALF), jnp.int32),
            pltpu.VMEM((2, SEQ, EMBED_DIM), jnp.float32),
            pltpu.VMEM((EMBED_DIM,), jnp.float32),
            pltpu.VMEM((16,), jnp.float32),
            pltpu.VMEM((BPW,), jnp.float32),
            pltpu.SemaphoreType.DMA,
            pltpu.SemaphoreType.DMA,
        ],
    )(_sc_body)
    b16 = jnp.broadcast_to(jnp.reshape(dense_b, (1,)), (16,))
    out = run(ids2, emb_table, w_flat, b16)
    return jnp.reshape(out, (BATCH, 1))


# speedup vs baseline: 78.5082x; 1.1654x over previous
import functools

import jax
import jax.numpy as jnp
from jax import lax
from jax.experimental import pallas as pl
from jax.experimental.pallas import tpu as pltpu
from jax.experimental.pallas import tpu_sc as plsc

EMBED_DIM = 128
BATCH = 4096
SEQ = 200
HALF = SEQ // 2
NC = 2
NS = 16
NW = NC * NS
BPW = BATCH // NW
NCHUNK = EMBED_DIM // 16


def _shuffle(v, idx):
    dn = lax.GatherDimensionNumbers(
        offset_dims=(), collapsed_slice_dims=(0,), start_index_map=(0,))
    return lax.gather(v, idx[:, None], dn, slice_sizes=(1,),
                      mode=lax.GatherScatterMode.PROMISE_IN_BOUNDS)


NBUF = 4


def _sc_body(ids_hbm, table_hbm, w_hbm, b_hbm, out_hbm,
             idx_v, rows_v, w_v, b_v, out_v, sem0, sem1, sem2, sem3):
    wid = lax.axis_index("s") * NC + lax.axis_index("c")
    base = wid * BPW
    hw = BPW // 2

    pltpu.sync_copy(w_hbm, w_v)
    pltpu.sync_copy(b_hbm, b_v)
    bias = b_v[...]
    w_chunks = [w_v[pl.ds(16 * c, 16)] for c in range(NCHUNK)]

    lane = lax.broadcasted_iota(jnp.int32, (16,), 0)
    sems = (sem0, sem1, sem2, sem3)

    def start_row(il, buf, sem):
        pltpu.async_copy(table_hbm.at[idx_v.at[2 * il]],
                         rows_v.at[buf, pl.ds(0, HALF)], sem)
        pltpu.async_copy(table_hbm.at[idx_v.at[2 * il + 1]],
                         rows_v.at[buf, pl.ds(HALF, HALF)], sem)

    def wait_row(il, buf, sem):
        pltpu.make_async_copy(table_hbm.at[idx_v.at[2 * il]],
                              rows_v.at[buf, pl.ds(0, HALF)], sem).wait()
        pltpu.make_async_copy(table_hbm.at[idx_v.at[2 * il + 1]],
                              rows_v.at[buf, pl.ds(HALF, HALF)], sem).wait()

    for h in range(2):
        hb = h * hw
        pltpu.sync_copy(ids_hbm.at[pl.ds((base + hb) * 2, 2 * hw)], idx_v)
        for p in range(NBUF - 1):
            start_row(p, p, sems[p])

        def group_body(g, carry):
            out_vec = jnp.full((16,), 0.0, dtype=jnp.float32)
            for j in range(16):
                il = g * 16 + j
                buf, nbuf = j % NBUF, (j + NBUF - 1) % NBUF

                @pl.when(il + NBUF - 1 < hw)
                def _():
                    start_row(il + NBUF - 1, nbuf, sems[nbuf])

                wait_row(il, buf, sems[buf])

                def red_body(r, m):
                    return tuple(
                        jnp.maximum(m[c], rows_v[buf, r, pl.ds(16 * c, 16)])
                        for c in range(NCHUNK))

                neg = jnp.full((16,), -jnp.inf, dtype=jnp.float32)
                m = lax.fori_loop(0, SEQ, red_body, (neg,) * NCHUNK,
                                  unroll=8)

                p = m[0] * w_chunks[0]
                for c in range(1, NCHUNK):
                    p = p + m[c] * w_chunks[c]
                for s in (8, 4, 2, 1):
                    p = p + _shuffle(p, lane ^ s)
                out_vec = jnp.where(lane == j, p + bias, out_vec)
            out_v[pl.ds(hb + g * 16, 16)] = out_vec
            return carry

        lax.fori_loop(0, hw // 16, group_body, 0)

    pltpu.sync_copy(out_v, out_hbm.at[pl.ds(base, BPW)])


def kernel(input_word_ids, emb_table, dense_w, dense_b):
    ids2 = jnp.reshape(input_word_ids.astype(jnp.int32), (2 * BATCH, HALF))
    w_flat = jnp.reshape(dense_w, (EMBED_DIM,))
    mesh = plsc.VectorSubcoreMesh(core_axis_name="c", subcore_axis_name="s")
    run = functools.partial(
        pl.kernel,
        mesh=mesh,
        out_type=jax.ShapeDtypeStruct((BATCH,), jnp.float32),
        scratch_types=[
            pltpu.VMEM((BPW, HALF), jnp.int32),
            pltpu.VMEM((NBUF, SEQ, EMBED_DIM), jnp.float32),
            pltpu.VMEM((EMBED_DIM,), jnp.float32),
            pltpu.VMEM((16,), jnp.float32),
            pltpu.VMEM((BPW,), jnp.float32),
            pltpu.SemaphoreType.DMA,
            pltpu.SemaphoreType.DMA,
            pltpu.SemaphoreType.DMA,
            pltpu.SemaphoreType.DMA,
        ],
    )(_sc_body)
    b16 = jnp.broadcast_to(jnp.reshape(dense_b, (1,)), (16,))
    out = run(ids2, emb_table, w_flat, b16)
    return jnp.reshape(out, (BATCH, 1))


# speedup vs baseline: 80.2643x; 1.0224x over previous
import functools

import jax
import jax.numpy as jnp
from jax import lax
from jax.experimental import pallas as pl
from jax.experimental.pallas import tpu as pltpu
from jax.experimental.pallas import tpu_sc as plsc

EMBED_DIM = 128
BATCH = 4096
SEQ = 200
HALF = SEQ // 2
NC = 2
NS = 16
NW = NC * NS
BPW = BATCH // NW
NCHUNK = EMBED_DIM // 16


def _shuffle(v, idx):
    dn = lax.GatherDimensionNumbers(
        offset_dims=(), collapsed_slice_dims=(0,), start_index_map=(0,))
    return lax.gather(v, idx[:, None], dn, slice_sizes=(1,),
                      mode=lax.GatherScatterMode.PROMISE_IN_BOUNDS)


NBUF = 4


def _sc_body(ids_hbm, table_hbm, w_hbm, b_hbm, out_hbm,
             idx_v, rows_v, w_v, b_v, out_v, sem0, sem1, sem2, sem3):
    wid = lax.axis_index("s") * NC + lax.axis_index("c")
    base = wid * BPW
    hw = BPW // 2

    pltpu.sync_copy(w_hbm, w_v)
    pltpu.sync_copy(b_hbm, b_v)
    bias = b_v[...]
    w_chunks = [w_v[pl.ds(16 * c, 16)] for c in range(NCHUNK)]

    lane = lax.broadcasted_iota(jnp.int32, (16,), 0)
    sems = (sem0, sem1, sem2, sem3)

    def start_row(il, buf, sem):
        pltpu.async_copy(table_hbm.at[idx_v.at[2 * il]],
                         rows_v.at[buf, pl.ds(0, HALF)], sem)
        pltpu.async_copy(table_hbm.at[idx_v.at[2 * il + 1]],
                         rows_v.at[buf, pl.ds(HALF, HALF)], sem)

    def wait_row(il, buf, sem):
        pltpu.make_async_copy(table_hbm.at[idx_v.at[2 * il]],
                              rows_v.at[buf, pl.ds(0, HALF)], sem).wait()
        pltpu.make_async_copy(table_hbm.at[idx_v.at[2 * il + 1]],
                              rows_v.at[buf, pl.ds(HALF, HALF)], sem).wait()

    for h in range(2):
        hb = h * hw
        pltpu.sync_copy(ids_hbm.at[pl.ds((base + hb) * 2, 2 * hw)], idx_v)
        for p in range(NBUF - 1):
            start_row(p, p, sems[p])

        def group_body(g, carry):
            out_vec = jnp.full((16,), 0.0, dtype=jnp.float32)
            for j in range(16):
                il = g * 16 + j
                buf, nbuf = j % NBUF, (j + NBUF - 1) % NBUF

                @pl.when(il + NBUF - 1 < hw)
                def _():
                    start_row(il + NBUF - 1, nbuf, sems[nbuf])

                wait_row(il, buf, sems[buf])

                neg = jnp.full((16,), -jnp.inf, dtype=jnp.float32)

                @plsc.parallel_loop(0, SEQ, unroll=8, carry=(neg,) * NCHUNK)
                def m(r, mc):
                    return tuple(
                        jnp.maximum(mc[c], rows_v[buf, r, pl.ds(16 * c, 16)])
                        for c in range(NCHUNK))

                p = m[0] * w_chunks[0]
                for c in range(1, NCHUNK):
                    p = p + m[c] * w_chunks[c]
                for s in (8, 4, 2, 1):
                    p = p + _shuffle(p, lane ^ s)
                out_vec = jnp.where(lane == j, p + bias, out_vec)
            out_v[pl.ds(hb + g * 16, 16)] = out_vec
            return carry

        lax.fori_loop(0, hw // 16, group_body, 0)

    pltpu.sync_copy(out_v, out_hbm.at[pl.ds(base, BPW)])


def kernel(input_word_ids, emb_table, dense_w, dense_b):
    ids2 = jnp.reshape(input_word_ids.astype(jnp.int32), (2 * BATCH, HALF))
    w_flat = jnp.reshape(dense_w, (EMBED_DIM,))
    mesh = plsc.VectorSubcoreMesh(core_axis_name="c", subcore_axis_name="s")
    run = functools.partial(
        pl.kernel,
        mesh=mesh,
        out_type=jax.ShapeDtypeStruct((BATCH,), jnp.float32),
        scratch_types=[
            pltpu.VMEM((BPW, HALF), jnp.int32),
            pltpu.VMEM((NBUF, SEQ, EMBED_DIM), jnp.float32),
            pltpu.VMEM((EMBED_DIM,), jnp.float32),
            pltpu.VMEM((16,), jnp.float32),
            pltpu.VMEM((BPW,), jnp.float32),
            pltpu.SemaphoreType.DMA,
            pltpu.SemaphoreType.DMA,
            pltpu.SemaphoreType.DMA,
            pltpu.SemaphoreType.DMA,
        ],
    )(_sc_body)
    b16 = jnp.broadcast_to(jnp.reshape(dense_b, (1,)), (16,))
    out = run(ids2, emb_table, w_flat, b16)
    return jnp.reshape(out, (BATCH, 1))


# speedup vs baseline: 80.3407x; 1.0010x over previous
import functools

import jax
import jax.numpy as jnp
from jax import lax
from jax.experimental import pallas as pl
from jax.experimental.pallas import tpu as pltpu
from jax.experimental.pallas import tpu_sc as plsc

EMBED_DIM = 128
BATCH = 4096
SEQ = 200
HALF = SEQ // 2
NC = 2
NS = 16
NW = NC * NS
BPW = BATCH // NW
NCHUNK = EMBED_DIM // 16


def _round_bf16(x):
    c = x * jnp.float32(65537.0)
    return c - (c - x)


def _shuffle(v, idx):
    dn = lax.GatherDimensionNumbers(
        offset_dims=(), collapsed_slice_dims=(0,), start_index_map=(0,))
    return lax.gather(v, idx[:, None], dn, slice_sizes=(1,),
                      mode=lax.GatherScatterMode.PROMISE_IN_BOUNDS)


NBUF = 4


def _sc_body(ids_hbm, table_hbm, w_hbm, b_hbm, out_hbm,
             idx_v, rows_v, w_v, b_v, out_v, sem0, sem1, sem2, sem3):
    wid = lax.axis_index("s") * NC + lax.axis_index("c")
    base = wid * BPW
    hw = BPW // 2

    pltpu.sync_copy(w_hbm, w_v)
    pltpu.sync_copy(b_hbm, b_v)
    bias = b_v[...]
    w_chunks = [_round_bf16(w_v[pl.ds(16 * c, 16)]) for c in range(NCHUNK)]

    lane = lax.broadcasted_iota(jnp.int32, (16,), 0)
    sems = (sem0, sem1, sem2, sem3)

    def start_row(il, buf, sem):
        pltpu.async_copy(table_hbm.at[idx_v.at[2 * il]],
                         rows_v.at[buf, pl.ds(0, HALF)], sem)
        pltpu.async_copy(table_hbm.at[idx_v.at[2 * il + 1]],
                         rows_v.at[buf, pl.ds(HALF, HALF)], sem)

    def wait_row(il, buf, sem):
        pltpu.make_async_copy(table_hbm.at[idx_v.at[2 * il]],
                              rows_v.at[buf, pl.ds(0, HALF)], sem).wait()
        pltpu.make_async_copy(table_hbm.at[idx_v.at[2 * il + 1]],
                              rows_v.at[buf, pl.ds(HALF, HALF)], sem).wait()

    for h in range(2):
        hb = h * hw
        pltpu.sync_copy(ids_hbm.at[pl.ds((base + hb) * 2, 2 * hw)], idx_v)
        for p in range(NBUF - 1):
            start_row(p, p, sems[p])

        def group_body(g, carry):
            out_vec = jnp.full((16,), 0.0, dtype=jnp.float32)
            for j in range(16):
                il = g * 16 + j
                buf, nbuf = j % NBUF, (j + NBUF - 1) % NBUF

                @pl.when(il + NBUF - 1 < hw)
                def _():
                    start_row(il + NBUF - 1, nbuf, sems[nbuf])

                wait_row(il, buf, sems[buf])

                neg = jnp.full((16,), -jnp.inf, dtype=jnp.float32)

                @plsc.parallel_loop(0, SEQ, unroll=8, carry=(neg,) * NCHUNK)
                def m(r, mc):
                    return tuple(
                        jnp.maximum(mc[c], rows_v[buf, r, pl.ds(16 * c, 16)])
                        for c in range(NCHUNK))

                p = _round_bf16(m[0]) * w_chunks[0]
                for c in range(1, NCHUNK):
                    p = p + _round_bf16(m[c]) * w_chunks[c]
                for s in (8, 4, 2, 1):
                    p = p + _shuffle(p, lane ^ s)
                out_vec = jnp.where(lane == j, p + bias, out_vec)
            out_v[pl.ds(hb + g * 16, 16)] = out_vec
            return carry

        lax.fori_loop(0, hw // 16, group_body, 0)

    pltpu.sync_copy(out_v, out_hbm.at[pl.ds(base, BPW)])


def kernel(input_word_ids, emb_table, dense_w, dense_b):
    ids2 = jnp.reshape(input_word_ids.astype(jnp.int32), (2 * BATCH, HALF))
    w_flat = jnp.reshape(dense_w, (EMBED_DIM,))
    mesh = plsc.VectorSubcoreMesh(core_axis_name="c", subcore_axis_name="s")
    run = functools.partial(
        pl.kernel,
        mesh=mesh,
        out_type=jax.ShapeDtypeStruct((BATCH,), jnp.float32),
        scratch_types=[
            pltpu.VMEM((BPW, HALF), jnp.int32),
            pltpu.VMEM((NBUF, SEQ, EMBED_DIM), jnp.float32),
            pltpu.VMEM((EMBED_DIM,), jnp.float32),
            pltpu.VMEM((16,), jnp.float32),
            pltpu.VMEM((BPW,), jnp.float32),
            pltpu.SemaphoreType.DMA,
            pltpu.SemaphoreType.DMA,
            pltpu.SemaphoreType.DMA,
            pltpu.SemaphoreType.DMA,
        ],
    )(_sc_body)
    b16 = jnp.broadcast_to(jnp.reshape(dense_b, (1,)), (16,))
    out = run(ids2, emb_table, w_flat, b16)
    return jnp.reshape(out, (BATCH, 1))
